# Initial kernel scaffold; baseline (speedup 1.0000x reference)
import functools

import jax
import jax.numpy as jnp
from jax import lax
from jax.experimental import pallas as pl
from jax.experimental.pallas import tpu as pltpu
from jax.experimental.pallas import tpu_sc as plsc

NUM_ELEMENTS = 118
EMBED_DIM = 64

_NC = 2
_NS = 16
_NW = _NC * _NS


def _make_embed(B: int, C: int):
    assert B % (_NW * C) == 0
    per_w = B // _NW
    n_chunks = per_w // C
    mesh = plsc.VectorSubcoreMesh(core_axis_name="c", subcore_axis_name="s")

    @functools.partial(
        pl.kernel,
        mesh=mesh,
        out_type=jax.ShapeDtypeStruct((B, EMBED_DIM), jnp.float32),
        scratch_types=[
            pltpu.VMEM((C,), jnp.int32),
            pltpu.VMEM((C, EMBED_DIM), jnp.float32),
            pltpu.SemaphoreType.DMA,
        ],
    )
    def emb(table_hbm, idx_hbm, out_hbm, idx_v, rows_v, sem):
        wid = lax.axis_index("s") * _NC + lax.axis_index("c")
        base = wid * per_w

        def body(i, carry):
            off = base + i * C
            pltpu.sync_copy(idx_hbm.at[pl.ds(off, C)], idx_v)
            pltpu.async_copy(table_hbm.at[idx_v], rows_v, sem).wait()
            pltpu.sync_copy(rows_v, out_hbm.at[pl.ds(off, C)])
            return carry

        lax.fori_loop(0, n_chunks, body, 0)

    return emb


def kernel(z, table):
    n, m = z.shape
    zf = z.reshape(n * m).astype(jnp.int32)
    out = _make_embed(n * m, 128)(table, zf)
    return out.reshape(n, m, EMBED_DIM)


# speedup vs baseline: 3.0354x; 3.0354x over previous
import functools

import jax
import jax.numpy as jnp
from jax import lax
from jax.experimental import pallas as pl
from jax.experimental.pallas import tpu as pltpu
from jax.experimental.pallas import tpu_sc as plsc

NUM_ELEMENTS = 118
EMBED_DIM = 64

_NC = 2
_NS = 16
_NW = _NC * _NS


def _make_embed(B: int, C: int):
    assert B % (_NW * C) == 0
    per_w = B // _NW
    n_chunks = per_w // C
    mesh = plsc.VectorSubcoreMesh(core_axis_name="c", subcore_axis_name="s")

    @functools.partial(
        pl.kernel,
        mesh=mesh,
        out_type=jax.ShapeDtypeStruct((B, EMBED_DIM), jnp.float32),
        compiler_params=pltpu.CompilerParams(use_tc_tiling_on_sc=False),
        scratch_types=[
            pltpu.VMEM((C,), jnp.int32),
            pltpu.VMEM((C, EMBED_DIM), jnp.float32),
            pltpu.SemaphoreType.DMA,
        ],
    )
    def emb(table_hbm, idx_hbm, out_hbm, idx_v, rows_v, sem):
        wid = lax.axis_index("s") * _NC + lax.axis_index("c")
        base = wid * per_w

        def body(i, carry):
            off = base + i * C
            pltpu.sync_copy(idx_hbm.at[pl.ds(off, C)], idx_v)
            pltpu.async_copy(table_hbm.at[idx_v], rows_v, sem).wait()
            pltpu.sync_copy(rows_v, out_hbm.at[pl.ds(off, C)])
            return carry

        lax.fori_loop(0, n_chunks, body, 0)

    return emb


def kernel(z, table):
    n, m = z.shape
    zf = z.reshape(n * m).astype(jnp.int32)
    out = _make_embed(n * m, 128)(table, zf)
    return out.reshape(n, m, EMBED_DIM)


# speedup vs baseline: 3.2368x; 1.0664x over previous
import functools

import jax
import jax.numpy as jnp
from jax import lax
from jax.experimental import pallas as pl
from jax.experimental.pallas import tpu as pltpu
from jax.experimental.pallas import tpu_sc as plsc

NUM_ELEMENTS = 118
EMBED_DIM = 64

_NC = 2
_NS = 16
_NW = _NC * _NS


def _make_embed(B: int, C: int):
    assert B % (_NW * C) == 0 and C % 8 == 0
    per_w = B // _NW
    n_chunks = per_w // C
    assert n_chunks % 2 == 0
    mesh = plsc.VectorSubcoreMesh(core_axis_name="c", subcore_axis_name="s")

    @functools.partial(
        pl.kernel,
        mesh=mesh,
        out_type=jax.ShapeDtypeStruct((B, EMBED_DIM), jnp.float32),
        compiler_params=pltpu.CompilerParams(use_tc_tiling_on_sc=False),
        scratch_types=[
            pltpu.VMEM((C,), jnp.int32),
            pltpu.VMEM((C,), jnp.int32),
            pltpu.VMEM((C, EMBED_DIM), jnp.float32),
            pltpu.VMEM((C, EMBED_DIM), jnp.float32),
            pltpu.SemaphoreType.DMA,
            pltpu.SemaphoreType.DMA,
            pltpu.SemaphoreType.DMA,
            pltpu.SemaphoreType.DMA,
            pltpu.SemaphoreType.DMA,
        ],
    )
    def emb(table_hbm, idx_hbm, out_hbm, idx0, idx1, rows0, rows1,
            isem0, isem1, gsem, osem0, osem1):
        wid = lax.axis_index("s") * _NC + lax.axis_index("c")
        base = wid * per_w
        idx_bufs = (idx0, idx1)
        rows_bufs = (rows0, rows1)
        isems = (isem0, isem1)
        osems = (osem0, osem1)

        def idx_start(g, b):
            pltpu.async_copy(idx_hbm.at[pl.ds(base + g * C, C)], idx_bufs[b], isems[b])

        def idx_wait(b):
            pltpu.make_async_copy(idx_hbm.at[pl.ds(0, C)], idx_bufs[b], isems[b]).wait()

        def out_start(g, b):
            pltpu.async_copy(rows_bufs[b], out_hbm.at[pl.ds(base + g * C, C)], osems[b])

        def out_wait(b):
            pltpu.make_async_copy(rows_bufs[b], out_hbm.at[pl.ds(0, C)], osems[b]).wait()

        idx_start(0, 0)
        idx_start(1, 1)

        def body(i, carry):
            for b in (0, 1):
                g = 2 * i + b
                idx_wait(b)
                @pl.when(g >= 2)
                def _():
                    out_wait(b)
                pltpu.async_copy(table_hbm.at[idx_bufs[b]], rows_bufs[b], gsem).wait()
                @pl.when(g + 2 < n_chunks)
                def _():
                    idx_start(g + 2, b)
                out_start(g, b)
            return carry

        lax.fori_loop(0, n_chunks // 2, body, 0)
        out_wait(0)
        out_wait(1)

    return emb


def kernel(z, table):
    n, m = z.shape
    zf = z.reshape(n * m).astype(jnp.int32)
    out = _make_embed(n * m, 800)(table, zf)
    return out.reshape(n, m, EMBED_DIM)


# speedup vs baseline: 3.2398x; 1.0009x over previous
import functools

import jax
import jax.numpy as jnp
from jax import lax
from jax.experimental import pallas as pl
from jax.experimental.pallas import tpu as pltpu
from jax.experimental.pallas import tpu_sc as plsc

NUM_ELEMENTS = 118
EMBED_DIM = 64

_NC = 2
_NS = 16
_NW = _NC * _NS


def _make_embed(N: int, M: int, R: int):
    B = N * M
    C = R * M
    assert B % (_NW * C) == 0 and C % 8 == 0
    rows_per_w = N // _NW
    per_w = B // _NW
    n_chunks = per_w // C
    assert n_chunks % 2 == 0
    mesh = plsc.VectorSubcoreMesh(core_axis_name="c", subcore_axis_name="s")

    @functools.partial(
        pl.kernel,
        mesh=mesh,
        out_type=jax.ShapeDtypeStruct((N, M, EMBED_DIM), jnp.float32),
        compiler_params=pltpu.CompilerParams(use_tc_tiling_on_sc=False),
        scratch_types=[
            pltpu.VMEM((C,), jnp.int32),
            pltpu.VMEM((C,), jnp.int32),
            pltpu.VMEM((C, EMBED_DIM), jnp.float32),
            pltpu.VMEM((C, EMBED_DIM), jnp.float32),
            pltpu.SemaphoreType.DMA,
            pltpu.SemaphoreType.DMA,
            pltpu.SemaphoreType.DMA,
            pltpu.SemaphoreType.DMA,
            pltpu.SemaphoreType.DMA,
        ],
    )
    def emb(table_hbm, idx_hbm, out_hbm, idx0, idx1, rows0, rows1,
            isem0, isem1, gsem, osem0, osem1):
        wid = lax.axis_index("s") * _NC + lax.axis_index("c")
        base = wid * per_w
        row0 = wid * rows_per_w
        idx_bufs = (idx0, idx1)
        rows_bufs = (rows0, rows1)
        isems = (isem0, isem1)
        osems = (osem0, osem1)

        def idx_start(g, b):
            pltpu.async_copy(idx_hbm.at[pl.ds(base + g * C, C)], idx_bufs[b], isems[b])

        def idx_wait(b):
            pltpu.make_async_copy(idx_hbm.at[pl.ds(0, C)], idx_bufs[b], isems[b]).wait()

        def out_start(g, b):
            r = row0 + g * R
            for j in range(R):
                pltpu.async_copy(rows_bufs[b].at[pl.ds(j * M, M)],
                                 out_hbm.at[r + j], osems[b])

        def out_wait(b):
            for j in range(R):
                pltpu.make_async_copy(rows_bufs[b].at[pl.ds(j * M, M)],
                                      out_hbm.at[0], osems[b]).wait()

        idx_start(0, 0)
        idx_start(1, 1)

        def body(i, carry):
            for b in (0, 1):
                g = 2 * i + b
                idx_wait(b)
                @pl.when(g >= 2)
                def _():
                    out_wait(b)
                pltpu.async_copy(table_hbm.at[idx_bufs[b]], rows_bufs[b], gsem).wait()
                @pl.when(g + 2 < n_chunks)
                def _():
                    idx_start(g + 2, b)
                out_start(g, b)
            return carry

        lax.fori_loop(0, n_chunks // 2, body, 0)
        out_wait(0)
        out_wait(1)

    return emb


def kernel(z, table):
    n, m = z.shape
    zf = z.reshape(n * m).astype(jnp.int32)
    return _make_embed(n, m, 4)(table, zf)
